# Initial kernel scaffold; baseline (speedup 1.0000x reference)
import jax
import jax.numpy as jnp
from jax.experimental import pallas as pl


def kernel(W_list, edge_index, w1a, b1a, w2a, b2a, eps1, w1b, b1b, w2b, b2b, eps2):
    raise NotImplementedError("write your pallas kernel here")


# speedup vs baseline: 28.3362x; 28.3362x over previous
import jax
import jax.numpy as jnp
from jax.experimental import pallas as pl
from jax.experimental.pallas import tpu as pltpu


def _adj_body(e_ref, eT_ref, a_ref):
    i = pl.program_id(0)
    n = a_ref.shape[0]
    ec = e_ref.shape[1]
    e = e_ref[...]
    eT = eT_ref[...]
    dst_row = e[1:2, :]
    src_col = eT[:, 0:1]
    iota_p = jax.lax.broadcasted_iota(jnp.int32, (n, ec), 0)
    ohd = (iota_p == dst_row).astype(jnp.bfloat16)
    iota_n = jax.lax.broadcasted_iota(jnp.int32, (ec, n), 1)
    ohs = (iota_n == src_col).astype(jnp.bfloat16)
    part = jnp.dot(ohd, ohs, preferred_element_type=jnp.float32)

    @pl.when(i == 0)
    def _():
        a_ref[...] = jnp.zeros_like(a_ref)

    a_ref[...] += part


def _gin_body(e1_ref, e2_ref, w1a_ref, w2a_ref, w1b_ref, b1a_ref, b2a_ref,
              b1b_ref, a_ref, x_ref, w2b_ref, b2b_ref, out_ref):
    i = pl.program_id(0)
    n = a_ref.shape[0]
    d_in = x_ref.shape[0]
    d_h = w1a_ref.shape[1]
    d_out = w2b_ref.shape[1]
    mb = x_ref.shape[2]
    f32 = jnp.float32

    a = a_ref[...]
    e1 = 1.0 + e1_ref[0, 0]
    e2 = 1.0 + e2_ref[0, 0]

    xs = [x_ref[d] for d in range(d_in)]
    xcat = jnp.concatenate(xs, axis=1)
    hcat = jnp.dot(a, xcat, preferred_element_type=f32)
    hs = [hcat[:, d * mb:(d + 1) * mb] + e1 * xs[d] for d in range(d_in)]

    t1 = [
        jax.nn.relu(
            sum(hs[d] * w1a_ref[d, f] for d in range(d_in)) + b1a_ref[0, f])
        for f in range(d_h)
    ]
    x1 = [
        jax.nn.relu(
            sum(t1[f] * w2a_ref[f, g] for f in range(d_h)) + b2a_ref[0, g])
        for g in range(d_h)
    ]

    x1cat = jnp.concatenate(x1, axis=1)
    h2cat = jnp.dot(a, x1cat, preferred_element_type=f32)
    hs2 = [h2cat[:, g * mb:(g + 1) * mb] + e2 * x1[g] for g in range(d_h)]

    t2 = [
        jax.nn.relu(
            sum(hs2[g] * w1b_ref[g, f] for g in range(d_h)) + b1b_ref[0, f])
        for f in range(d_h)
    ]

    rs = [jnp.sum(t2[f], axis=1, keepdims=True) for f in range(d_h)]
    pe = sum(rs[f] * w2b_ref[f:f + 1, :] for f in range(d_h))

    @pl.when(i == 0)
    def _():
        out_ref[...] = float(n) * jnp.broadcast_to(b2b_ref[...], (n, d_out))

    out_ref[...] += pe


def kernel(W_list, edge_index, w1a, b1a, w2a, b2a, eps1, w1b, b1b, w2b, b2b,
           eps2):
    n_graphs, n_max, n_nodes_dim, d_in = (W_list.shape[0], W_list.shape[1],
                                          W_list.shape[2], W_list.shape[3])
    n = n_graphs * n_max
    m = n_nodes_dim
    d_h = w1a.shape[1]
    d_out = w2b.shape[1]
    e_total = edge_index.shape[1]

    x0p = W_list.reshape(n, m, d_in).transpose(2, 0, 1)
    eT = edge_index.T

    ec = 1024
    n_chunks = e_total // ec
    adj = pl.pallas_call(
        _adj_body,
        grid=(n_chunks,),
        in_specs=[
            pl.BlockSpec((2, ec), lambda i: (0, i)),
            pl.BlockSpec((ec, 2), lambda i: (i, 0)),
        ],
        out_specs=pl.BlockSpec((n, n), lambda i: (0, 0)),
        out_shape=jax.ShapeDtypeStruct((n, n), jnp.float32),
    )(edge_index, eT)

    mb = 128
    grid = m // mb
    smem = pltpu.SMEM
    pe = pl.pallas_call(
        _gin_body,
        grid=(grid,),
        in_specs=[
            pl.BlockSpec(memory_space=smem),
            pl.BlockSpec(memory_space=smem),
            pl.BlockSpec(memory_space=smem),
            pl.BlockSpec(memory_space=smem),
            pl.BlockSpec(memory_space=smem),
            pl.BlockSpec(memory_space=smem),
            pl.BlockSpec(memory_space=smem),
            pl.BlockSpec(memory_space=smem),
            pl.BlockSpec((n, n), lambda i: (0, 0)),
            pl.BlockSpec((d_in, n, mb), lambda i: (0, 0, i)),
            pl.BlockSpec((d_h, d_out), lambda i: (0, 0)),
            pl.BlockSpec((1, d_out), lambda i: (0, 0)),
        ],
        out_specs=pl.BlockSpec((n, d_out), lambda i: (0, 0)),
        out_shape=jax.ShapeDtypeStruct((n, d_out), jnp.float32),
    )(
        eps1.reshape(1, 1), eps2.reshape(1, 1), w1a, w2a, w1b,
        b1a.reshape(1, d_h), b2a.reshape(1, d_h), b1b.reshape(1, d_h),
        adj, x0p, w2b, b2b.reshape(1, d_out),
    )
    return pe


# speedup vs baseline: 33.4748x; 1.1813x over previous
import jax
import jax.numpy as jnp
from jax.experimental import pallas as pl
from jax.experimental.pallas import tpu as pltpu


def _adj_body(e_ref, a_ref):
    i = pl.program_id(0)
    n = a_ref.shape[0]
    ec = e_ref.shape[1]
    e = e_ref[...]
    dst_row = e[1:2, :]
    src_row = e[0:1, :]
    iota_p = jax.lax.broadcasted_iota(jnp.int32, (n, ec), 0)
    ohd = (iota_p == dst_row).astype(jnp.bfloat16)
    ohs = (iota_p == src_row).astype(jnp.bfloat16)
    part = jax.lax.dot_general(
        ohd, ohs, (((1,), (1,)), ((), ())),
        preferred_element_type=jnp.float32)

    @pl.when(i == 0)
    def _():
        a_ref[...] = jnp.zeros_like(a_ref)

    a_ref[...] += part.astype(a_ref.dtype)


def _gin_body(e1_ref, e2_ref, w1a_ref, w2a_ref, w1b_ref, b1a_ref, b2a_ref,
              b1b_ref, a_ref, x_ref, w2b_ref, b2b_ref, out_ref):
    i = pl.program_id(0)
    n = a_ref.shape[0]
    d_in = x_ref.shape[0]
    d_h = w1a_ref.shape[1]
    d_out = w2b_ref.shape[1]
    mb = x_ref.shape[2]
    f32 = jnp.float32

    a = a_ref[...].astype(jnp.bfloat16)
    e1 = 1.0 + e1_ref[0, 0]
    e2 = 1.0 + e2_ref[0, 0]

    xs = [x_ref[d] for d in range(d_in)]
    xcat = jnp.concatenate(xs, axis=1)
    hcat = jnp.dot(a, xcat, preferred_element_type=f32)
    hs = [hcat[:, d * mb:(d + 1) * mb] + e1 * xs[d] for d in range(d_in)]

    t1 = [
        jax.nn.relu(
            sum(hs[d] * w1a_ref[d, f] for d in range(d_in)) + b1a_ref[0, f])
        for f in range(d_h)
    ]
    x1 = [
        jax.nn.relu(
            sum(t1[f] * w2a_ref[f, g] for f in range(d_h)) + b2a_ref[0, g])
        for g in range(d_h)
    ]

    x1cat = jnp.concatenate(x1, axis=1).astype(jnp.bfloat16)
    h2cat = jnp.dot(a, x1cat, preferred_element_type=f32)
    hs2 = [h2cat[:, g * mb:(g + 1) * mb] + e2 * x1[g] for g in range(d_h)]

    t2 = [
        jax.nn.relu(
            sum(hs2[g] * w1b_ref[g, f] for g in range(d_h)) + b1b_ref[0, f])
        for f in range(d_h)
    ]

    rs = [jnp.sum(t2[f], axis=1, keepdims=True) for f in range(d_h)]
    pe = sum(rs[f] * w2b_ref[f:f + 1, :] for f in range(d_h))

    @pl.when(i == 0)
    def _():
        out_ref[...] = float(n) * jnp.broadcast_to(b2b_ref[...], (n, d_out))

    out_ref[...] += pe


def kernel(W_list, edge_index, w1a, b1a, w2a, b2a, eps1, w1b, b1b, w2b, b2b,
           eps2):
    n_graphs, n_max, n_nodes_dim, d_in = (W_list.shape[0], W_list.shape[1],
                                          W_list.shape[2], W_list.shape[3])
    n = n_graphs * n_max
    m = n_nodes_dim
    d_h = w1a.shape[1]
    d_out = w2b.shape[1]
    e_total = edge_index.shape[1]

    x0p = W_list.reshape(n, m, d_in).astype(jnp.bfloat16).transpose(2, 0, 1)

    ec = 1024
    n_chunks = e_total // ec
    adj = pl.pallas_call(
        _adj_body,
        grid=(n_chunks,),
        in_specs=[
            pl.BlockSpec((2, ec), lambda i: (0, i)),
        ],
        out_specs=pl.BlockSpec((n, n), lambda i: (0, 0)),
        out_shape=jax.ShapeDtypeStruct((n, n), jnp.float32),
    )(edge_index)

    mb = 256
    grid = m // mb
    smem = pltpu.SMEM
    pe = pl.pallas_call(
        _gin_body,
        grid=(grid,),
        in_specs=[
            pl.BlockSpec(memory_space=smem),
            pl.BlockSpec(memory_space=smem),
            pl.BlockSpec(memory_space=smem),
            pl.BlockSpec(memory_space=smem),
            pl.BlockSpec(memory_space=smem),
            pl.BlockSpec(memory_space=smem),
            pl.BlockSpec(memory_space=smem),
            pl.BlockSpec(memory_space=smem),
            pl.BlockSpec((n, n), lambda i: (0, 0)),
            pl.BlockSpec((d_in, n, mb), lambda i: (0, 0, i)),
            pl.BlockSpec((d_h, d_out), lambda i: (0, 0)),
            pl.BlockSpec((1, d_out), lambda i: (0, 0)),
        ],
        out_specs=pl.BlockSpec((n, d_out), lambda i: (0, 0)),
        out_shape=jax.ShapeDtypeStruct((n, d_out), jnp.float32),
    )(
        eps1.reshape(1, 1), eps2.reshape(1, 1), w1a, w2a, w1b,
        b1a.reshape(1, d_h), b2a.reshape(1, d_h), b1b.reshape(1, d_h),
        adj, x0p, w2b, b2b.reshape(1, d_out),
    )
    return pe
